# baseline (device time: 131520 ns/iter reference)
import jax
import jax.numpy as jnp
from jax import lax
from jax.experimental import pallas as pl
from jax.experimental.pallas import tpu as pltpu

N_DEV = 4
N_TOK = 2048
D = 512
H = 1024
N_EXP = 32
E_PER = N_EXP // N_DEV
CAP = 51
PAD = 64
SLOTS_PER_DEV = E_PER * PAD
N_SLOTS = N_DEV * SLOTS_PER_DEV


def _moe_body(gathered_ref, ew_ref, out_ref, send_sems, recv_sems):
    my = lax.axis_index("i")
    left = lax.rem(my - 1 + N_DEV, N_DEV)
    right = lax.rem(my + 1, N_DEV)

    barrier_sem = pltpu.get_barrier_semaphore()
    for nbr in (left, right):
        pl.semaphore_signal(
            barrier_sem, inc=1,
            device_id=(nbr,), device_id_type=pl.DeviceIdType.MESH,
        )
    pl.semaphore_wait(barrier_sem, 2)

    for le in range(E_PER):
        rows = gathered_ref[pl.ds(le * PAD, PAD), :]
        h_le = jnp.dot(
            rows, ew_ref[le],
            precision=lax.Precision.HIGHEST,
            preferred_element_type=jnp.float32,
        )
        out_ref[pl.ds(my, 1), pl.ds(le * PAD, PAD), :] = h_le[None]

    for h in range(N_DEV - 1):
        slot = lax.rem(my - h + N_DEV, N_DEV)
        rdma = pltpu.make_async_remote_copy(
            src_ref=out_ref.at[slot],
            dst_ref=out_ref.at[slot],
            send_sem=send_sems.at[h],
            recv_sem=recv_sems.at[h],
            device_id=(right,),
            device_id_type=pl.DeviceIdType.MESH,
        )
        rdma.start()
        rdma.wait()


def kernel(x, router_W, route_idx, expert_W):
    del router_W

    my = lax.axis_index("i")

    e = route_idx[:, 0].astype(jnp.int32)
    onehot = (e[:, None] == jnp.arange(N_EXP, dtype=jnp.int32)[None, :])
    pos_all = jnp.cumsum(onehot.astype(jnp.int32), axis=0) - 1
    tok_pos = jnp.sum(jnp.where(onehot, pos_all, 0), axis=1)
    keep = tok_pos < CAP

    owner = e // E_PER
    le = e % E_PER
    slot = jnp.where(
        keep, owner * SLOTS_PER_DEV + le * PAD + tok_pos, N_SLOTS
    )
    token_ids = jnp.arange(N_TOK, dtype=jnp.int32)
    idx_all = (
        jnp.full((N_SLOTS + 1,), N_TOK, jnp.int32).at[slot].set(token_ids)
    )[:N_SLOTS]

    my_idx = lax.dynamic_slice(idx_all, (my * SLOTS_PER_DEV,), (SLOTS_PER_DEV,))
    x_pad = jnp.concatenate([x, jnp.zeros((1, D), jnp.float32)], axis=0)
    gathered = jnp.take(x_pad, my_idx, axis=0)

    comp_all = pl.pallas_call(
        _moe_body,
        out_shape=jax.ShapeDtypeStruct((N_DEV, SLOTS_PER_DEV, H), jnp.float32),
        in_specs=[
            pl.BlockSpec(memory_space=pltpu.VMEM),
            pl.BlockSpec(memory_space=pltpu.VMEM),
        ],
        out_specs=pl.BlockSpec(memory_space=pltpu.VMEM),
        scratch_shapes=[
            pltpu.SemaphoreType.DMA((N_DEV - 1,)),
            pltpu.SemaphoreType.DMA((N_DEV - 1,)),
        ],
        compiler_params=pltpu.CompilerParams(collective_id=0),
    )(gathered, expert_W)

    out = (
        jnp.zeros((N_TOK + 1, H), jnp.float32)
        .at[idx_all]
        .set(comp_all.reshape(N_SLOTS, H))
    )[:N_TOK]
    return out
